# baseline (speedup 1.0000x reference)
import jax
import jax.numpy as jnp
from jax.experimental import pallas as pl
from jax.experimental.pallas import tpu as pltpu

_MM = (((1,), (0,)), ((), ()))
_TA = (((0,), (0,)), ((), ()))
_TT = (((0,), (1,)), ((), ()))


def _make_fused_kernel(H1, W1, C_in, H, W, H_up, W_up, oY, oX,
                       C_half, C_mid, C_out):
    Wx = W + 8
    band = H * Wx
    baseL = 128
    SxL = baseL + band + 128
    f32 = jnp.float32
    bf16 = jnp.bfloat16
    W1_OFF = 0
    W2_OFF = W1_OFF + 9 * C_mid
    WU_OFF = W2_OFF + 9 * C_out
    BT_OFF = WU_OFF + 2 * C_in
    EYE_OFF = BT_OFF + 2

    def body(x1_ref, x2_ref, wa_ref, fb_ref, mask_ref, o_ref,
             slab, hs, ps, ups):
        xb = x1_ref[0].astype(bf16)
        for di in range(2):
            wu = wa_ref[WU_OFF + di * C_in:WU_OFF + (di + 1) * C_in, :]
            bu = wa_ref[BT_OFF + di:BT_OFF + di + 1, :].astype(f32)
            ps[:, 2 * C_half * di:2 * C_half * (di + 1)] = (
                jax.lax.dot_general(xb, wu, dimension_numbers=_TA,
                                    preferred_element_type=f32) + bu)

        ups[...] = jnp.zeros((SxL, C_half), bf16)
        for r in range(H_up):
            i, di = r // 2, r % 2
            v = ps[W1 * i:W1 * i + W1, 2 * C_half * di:2 * C_half * (di + 1)]
            v = v.reshape(W_up, C_half)
            ups[pl.ds(baseL + (r + oY) * Wx + 4 + oX, W_up), :] = (
                v.astype(bf16))

        eye = wa_ref[EYE_OFF:EYE_OFF + C_half, 0:C_half]
        slab[0:C_half, :] = jnp.zeros((C_half, SxL), bf16)
        for r in range(H):
            slab[0:C_half, pl.ds(baseL + r * Wx + 4, W)] = (
                x2_ref[0][:, W * r:W * r + W].astype(bf16))
        slab[C_half:2 * C_half, :] = jax.lax.dot_general(
            eye, ups[...], dimension_numbers=_TT,
            preferred_element_type=f32).astype(bf16)

        b1 = fb_ref[0:C_mid, 0:1]
        acc = b1 + jnp.zeros((C_mid, band), f32)
        for dh in range(3):
            for dw in range(3):
                k = dh * 3 + dw
                off = baseL + (dh - 1) * Wx + (dw - 1)
                acc = acc + jax.lax.dot_general(
                    wa_ref[W1_OFF + k * C_mid:W1_OFF + (k + 1) * C_mid, :],
                    slab[:, pl.ds(off, band)],
                    dimension_numbers=_MM, preferred_element_type=f32)
        hval = jnp.maximum(acc, 0.0) * mask_ref[...]
        hs[:, 0:baseL] = jnp.zeros((C_mid, baseL), bf16)
        hs[:, baseL + band:SxL] = jnp.zeros((C_mid, SxL - band - baseL), bf16)
        hs[:, pl.ds(baseL, band)] = hval.astype(bf16)

        b2 = fb_ref[C_mid:C_mid + C_out, 0:1]
        acc2 = b2 + jnp.zeros((C_out, band), f32)
        for dh in range(3):
            for dw in range(3):
                k = dh * 3 + dw
                off = baseL + (dh - 1) * Wx + (dw - 1)
                acc2 = acc2 + jax.lax.dot_general(
                    wa_ref[W2_OFF + k * C_out:W2_OFF + (k + 1) * C_out,
                           0:C_mid],
                    hs[:, pl.ds(off, band)],
                    dimension_numbers=_MM, preferred_element_type=f32)
        oval = jnp.maximum(acc2, 0.0)
        for r in range(H):
            o_ref[0, :, pl.ds(W * r, W)] = oval[:, r * Wx + 4:r * Wx + 4 + W]

    return body, Wx, band, SxL, (W1_OFF, W2_OFF, WU_OFF, BT_OFF)


def kernel(x1, x2, wt, bt, w1, b1, g1, be1, m1, v1, w2, b2, g2, be2, m2, v2):
    N, C_in, H1, W1 = x1.shape
    _, C_half, H, W = x2.shape
    H_up, W_up = 2 * H1, 2 * W1
    oY, oX = (H - H_up) // 2, (W - W_up) // 2
    f32, bf16 = jnp.float32, jnp.bfloat16

    eps = 1e-5
    s1 = g1 / jnp.sqrt(v1 + eps)
    w1e = w1 * s1[:, None, None, None]
    b1e = (b1 - m1) * s1 + be1
    s2 = g2 / jnp.sqrt(v2 + eps)
    w2e = w2 * s2[:, None, None, None]
    b2e = (b2 - m2) * s2 + be2
    C_mid, C_cat = w1e.shape[0], w1e.shape[1]
    C_out = w2e.shape[0]

    body, Wx, band, SxL, offs = _make_fused_kernel(
        H1, W1, C_in, H, W, H_up, W_up, oY, oX, C_half, C_mid, C_out)

    w1r = jnp.transpose(w1e, (2, 3, 0, 1)).reshape(9 * C_mid, C_cat)
    w2r = jnp.transpose(w2e, (2, 3, 0, 1)).reshape(9 * C_out, C_mid)
    w2r = jnp.pad(w2r, ((0, 0), (0, C_cat - C_mid)))
    wun = jnp.transpose(wt, (0, 2, 3, 1)).reshape(C_in, 2, 2 * C_half)
    wun = jnp.transpose(wun, (1, 0, 2)).reshape(2 * C_in, 2 * C_half)
    bt4 = jnp.tile(bt, 4).reshape(2, 2 * C_half)
    eye = jnp.pad(jnp.eye(C_half, dtype=f32), ((0, 0), (0, C_half)))
    wa = jnp.concatenate([w1r, w2r, wun, bt4, eye], axis=0).astype(bf16)
    pad_rows = (-wa.shape[0]) % 8
    wa = jnp.pad(wa, ((0, pad_rows), (0, 0)))

    fb = jnp.concatenate([b1e, b2e], axis=0).reshape(C_mid + C_out, 1)
    fb = jnp.tile(fb, (1, 128)).astype(f32)

    x1r = x1.reshape(N, C_in, H1 * W1)
    x2r = x2.reshape(N, C_half, H * W)

    col = jnp.arange(band, dtype=jnp.int32) % Wx
    mask = ((col >= 4) & (col < 4 + W)).astype(f32).reshape(1, band)

    smp = lambda n: (n, 0, 0)
    cst2 = lambda n: (0, 0)
    out_flat = pl.pallas_call(
        body,
        out_shape=jax.ShapeDtypeStruct((N, C_out, H * W), f32),
        grid=(N,),
        in_specs=[
            pl.BlockSpec((1, C_in, H1 * W1), smp),
            pl.BlockSpec((1, C_half, H * W), smp),
            pl.BlockSpec(wa.shape, cst2),
            pl.BlockSpec(fb.shape, cst2),
            pl.BlockSpec((1, band), cst2),
        ],
        out_specs=pl.BlockSpec((1, C_out, H * W), smp),
        scratch_shapes=[
            pltpu.VMEM((2 * C_half, SxL), bf16),
            pltpu.VMEM((C_mid, SxL), bf16),
            pltpu.VMEM((H1 * W1, 4 * C_half), f32),
            pltpu.VMEM((SxL, C_half), bf16),
        ],
        compiler_params=pltpu.CompilerParams(
            dimension_semantics=("parallel",),
            vmem_limit_bytes=64 * 1024 * 1024),
    )(x1r, x2r, wa, fb, mask)

    return out_flat.reshape(N, C_out, H, W)


# speedup vs baseline: 1.0058x; 1.0058x over previous
import jax
import jax.numpy as jnp
from jax.experimental import pallas as pl
from jax.experimental.pallas import tpu as pltpu

_MM = (((1,), (0,)), ((), ()))
_TA = (((0,), (0,)), ((), ()))
_TT = (((0,), (1,)), ((), ()))


def _make_fused_kernel(H1, W1, C_in, H, W, H_up, W_up, oY, oX,
                       C_half, C_mid, C_out):
    Wx = W + 8
    band = H * Wx
    baseL = 128
    SxL = baseL + band + 128
    f32 = jnp.float32
    bf16 = jnp.bfloat16
    W1_OFF = 0
    W2_OFF = W1_OFF + 9 * C_mid
    WU_OFF = W2_OFF + 9 * C_out
    BT_OFF = WU_OFF + 2 * C_in
    EYE_OFF = BT_OFF + 2

    def body(x1_ref, x2_ref, wa_ref, fb_ref, o_ref,
             slab, hs, ps, ups):
        xb = x1_ref[0].astype(bf16)
        for di in range(2):
            wu = wa_ref[WU_OFF + di * C_in:WU_OFF + (di + 1) * C_in, :]
            bu = wa_ref[BT_OFF + di:BT_OFF + di + 1, :].astype(f32)
            ps[:, 2 * C_half * di:2 * C_half * (di + 1)] = (
                jax.lax.dot_general(xb, wu, dimension_numbers=_TA,
                                    preferred_element_type=f32) + bu)

        ups[...] = jnp.zeros((SxL, C_half), bf16)
        for r in range(H_up):
            i, di = r // 2, r % 2
            v = ps[W1 * i:W1 * i + W1, 2 * C_half * di:2 * C_half * (di + 1)]
            v = v.reshape(W_up, C_half)
            ups[pl.ds(baseL + (r + oY) * Wx + 4 + oX, W_up), :] = (
                v.astype(bf16))

        eye = wa_ref[EYE_OFF:EYE_OFF + C_half, 0:C_half]
        slab[0:C_half, :] = jnp.zeros((C_half, SxL), bf16)
        for r in range(H):
            slab[0:C_half, pl.ds(baseL + r * Wx + 4, W)] = (
                x2_ref[0][:, W * r:W * r + W].astype(bf16))
        slab[C_half:2 * C_half, :] = jax.lax.dot_general(
            eye, ups[...], dimension_numbers=_TT,
            preferred_element_type=f32).astype(bf16)

        b1 = fb_ref[0:C_mid, 0:1]
        acc = b1 + jnp.zeros((C_mid, band), f32)
        for dh in range(3):
            for dw in range(3):
                k = dh * 3 + dw
                off = baseL + (dh - 1) * Wx + (dw - 1)
                acc = acc + jax.lax.dot_general(
                    wa_ref[W1_OFF + k * C_mid:W1_OFF + (k + 1) * C_mid, :],
                    slab[:, pl.ds(off, band)],
                    dimension_numbers=_MM, preferred_element_type=f32)
        col = jax.lax.broadcasted_iota(jnp.int32, (1, band), 1) % Wx
        mask = ((col >= 4) & (col < 4 + W)).astype(f32)
        hval = jnp.maximum(acc, 0.0) * mask
        hs[:, 0:baseL] = jnp.zeros((C_mid, baseL), bf16)
        hs[:, baseL + band:SxL] = jnp.zeros((C_mid, SxL - band - baseL), bf16)
        hs[:, pl.ds(baseL, band)] = hval.astype(bf16)

        b2 = fb_ref[C_mid:C_mid + C_out, 0:1]
        acc2 = b2 + jnp.zeros((C_out, band), f32)
        for dh in range(3):
            for dw in range(3):
                k = dh * 3 + dw
                off = baseL + (dh - 1) * Wx + (dw - 1)
                acc2 = acc2 + jax.lax.dot_general(
                    wa_ref[W2_OFF + k * C_out:W2_OFF + (k + 1) * C_out,
                           0:C_mid],
                    hs[:, pl.ds(off, band)],
                    dimension_numbers=_MM, preferred_element_type=f32)
        oval = jnp.maximum(acc2, 0.0)
        for r in range(H):
            o_ref[0, :, pl.ds(W * r, W)] = oval[:, r * Wx + 4:r * Wx + 4 + W]

    return body, Wx, band, SxL, (W1_OFF, W2_OFF, WU_OFF, BT_OFF)


def kernel(x1, x2, wt, bt, w1, b1, g1, be1, m1, v1, w2, b2, g2, be2, m2, v2):
    N, C_in, H1, W1 = x1.shape
    _, C_half, H, W = x2.shape
    H_up, W_up = 2 * H1, 2 * W1
    oY, oX = (H - H_up) // 2, (W - W_up) // 2
    f32, bf16 = jnp.float32, jnp.bfloat16

    eps = 1e-5
    s1 = g1 / jnp.sqrt(v1 + eps)
    w1e = w1 * s1[:, None, None, None]
    b1e = (b1 - m1) * s1 + be1
    s2 = g2 / jnp.sqrt(v2 + eps)
    w2e = w2 * s2[:, None, None, None]
    b2e = (b2 - m2) * s2 + be2
    C_mid, C_cat = w1e.shape[0], w1e.shape[1]
    C_out = w2e.shape[0]

    body, Wx, band, SxL, offs = _make_fused_kernel(
        H1, W1, C_in, H, W, H_up, W_up, oY, oX, C_half, C_mid, C_out)

    w1r = jnp.transpose(w1e, (2, 3, 0, 1)).reshape(9 * C_mid, C_cat)
    w2r = jnp.transpose(w2e, (2, 3, 0, 1)).reshape(9 * C_out, C_mid)
    w2r = jnp.pad(w2r, ((0, 0), (0, C_cat - C_mid)))
    wun = jnp.transpose(wt, (0, 2, 3, 1)).reshape(C_in, 2, 2 * C_half)
    wun = jnp.transpose(wun, (1, 0, 2)).reshape(2 * C_in, 2 * C_half)
    bt4 = jnp.tile(bt, 4).reshape(2, 2 * C_half)
    eye = jnp.pad(jnp.eye(C_half, dtype=f32), ((0, 0), (0, C_half)))
    wa = jnp.concatenate([w1r, w2r, wun, bt4, eye], axis=0).astype(bf16)
    pad_rows = (-wa.shape[0]) % 8
    wa = jnp.pad(wa, ((0, pad_rows), (0, 0)))

    fb = jnp.concatenate([b1e, b2e], axis=0).reshape(C_mid + C_out, 1)
    fb = jnp.tile(fb, (1, 128)).astype(f32)

    x1r = x1.reshape(N, C_in, H1 * W1)
    x2r = x2.reshape(N, C_half, H * W)

    smp = lambda n: (n, 0, 0)
    cst2 = lambda n: (0, 0)
    out_flat = pl.pallas_call(
        body,
        out_shape=jax.ShapeDtypeStruct((N, C_out, H * W), f32),
        grid=(N,),
        in_specs=[
            pl.BlockSpec((1, C_in, H1 * W1), smp),
            pl.BlockSpec((1, C_half, H * W), smp),
            pl.BlockSpec(wa.shape, cst2),
            pl.BlockSpec(fb.shape, cst2),
        ],
        out_specs=pl.BlockSpec((1, C_out, H * W), smp),
        scratch_shapes=[
            pltpu.VMEM((2 * C_half, SxL), bf16),
            pltpu.VMEM((C_mid, SxL), bf16),
            pltpu.VMEM((H1 * W1, 4 * C_half), f32),
            pltpu.VMEM((SxL, C_half), bf16),
        ],
        compiler_params=pltpu.CompilerParams(
            dimension_semantics=("parallel",),
            vmem_limit_bytes=64 * 1024 * 1024),
    )(x1r, x2r, wa, fb)

    return out_flat.reshape(N, C_out, H, W)


# speedup vs baseline: 1.0141x; 1.0083x over previous
import jax
import jax.numpy as jnp
from jax.experimental import pallas as pl
from jax.experimental.pallas import tpu as pltpu

_MM = (((1,), (0,)), ((), ()))
_TA = (((0,), (0,)), ((), ()))
_TT = (((0,), (1,)), ((), ()))


def _make_fused_kernel(H1, W1, C_in, H, W, H_up, W_up, oY, oX,
                       C_half, C_mid, C_out):
    Wx = W + 8
    band = H * Wx
    baseL = 128
    SxL = baseL + band + 128
    f32 = jnp.float32
    bf16 = jnp.bfloat16
    W1_OFF = 0
    W2_OFF = W1_OFF + 9 * C_mid
    WU_OFF = W2_OFF + 9 * C_out
    BT_OFF = WU_OFF + 2 * C_in
    EYE_OFF = BT_OFF + 2

    def body(x1_ref, x2_ref, wa_ref, fb_ref, o_ref,
             slab, hs, ps, ups):
        xb = x1_ref[0].astype(bf16)
        for di in range(2):
            wu = wa_ref[WU_OFF + di * C_in:WU_OFF + (di + 1) * C_in, :]
            bu = wa_ref[BT_OFF + di:BT_OFF + di + 1, :].astype(f32)
            ps[:, 2 * C_half * di:2 * C_half * (di + 1)] = (
                jax.lax.dot_general(xb, wu, dimension_numbers=_TA,
                                    preferred_element_type=f32) + bu)

        ups[...] = jnp.zeros((SxL, C_half), bf16)
        for r in range(H_up):
            i, di = r // 2, r % 2
            v = ps[W1 * i:W1 * i + W1, 2 * C_half * di:2 * C_half * (di + 1)]
            v = v.reshape(W_up, C_half)
            ups[pl.ds(baseL + (r + oY) * Wx + 4 + oX, W_up), :] = (
                v.astype(bf16))

        eyei = jax.lax.broadcasted_iota(jnp.int32, (C_half, C_half), 0)
        eyej = jax.lax.broadcasted_iota(jnp.int32, (C_half, C_half), 1)
        eye = jnp.where(eyei == eyej, 1.0, 0.0).astype(bf16)
        slab[0:C_half, :] = jnp.zeros((C_half, SxL), bf16)
        for r in range(H):
            slab[0:C_half, pl.ds(baseL + r * Wx + 4, W)] = (
                x2_ref[0][:, W * r:W * r + W].astype(bf16))
        slab[C_half:2 * C_half, :] = jax.lax.dot_general(
            eye, ups[...], dimension_numbers=_TT,
            preferred_element_type=f32).astype(bf16)

        b1 = fb_ref[0:C_mid, 0:1]
        acc = b1 + jnp.zeros((C_mid, band), f32)
        for dh in range(3):
            for dw in range(3):
                k = dh * 3 + dw
                off = baseL + (dh - 1) * Wx + (dw - 1)
                acc = acc + jax.lax.dot_general(
                    wa_ref[W1_OFF + k * C_mid:W1_OFF + (k + 1) * C_mid, :],
                    slab[:, pl.ds(off, band)],
                    dimension_numbers=_MM, preferred_element_type=f32)
        col = jax.lax.broadcasted_iota(jnp.int32, (1, band), 1) % Wx
        mask = ((col >= 4) & (col < 4 + W)).astype(f32)
        hval = jnp.maximum(acc, 0.0) * mask
        hs[:, 0:baseL] = jnp.zeros((C_mid, baseL), bf16)
        hs[:, baseL + band:SxL] = jnp.zeros((C_mid, SxL - band - baseL), bf16)
        hs[:, pl.ds(baseL, band)] = hval.astype(bf16)

        b2 = fb_ref[C_mid:C_mid + C_out, 0:1]
        acc2 = b2 + jnp.zeros((C_out, band), f32)
        for dh in range(3):
            for dw in range(3):
                k = dh * 3 + dw
                off = baseL + (dh - 1) * Wx + (dw - 1)
                acc2 = acc2 + jax.lax.dot_general(
                    wa_ref[W2_OFF + k * C_out:W2_OFF + (k + 1) * C_out,
                           0:C_mid],
                    hs[:, pl.ds(off, band)],
                    dimension_numbers=_MM, preferred_element_type=f32)
        oval = jnp.maximum(acc2, 0.0)
        for r in range(H):
            o_ref[0, :, pl.ds(W * r, W)] = oval[:, r * Wx + 4:r * Wx + 4 + W]

    return body, Wx, band, SxL, (W1_OFF, W2_OFF, WU_OFF, BT_OFF)


def kernel(x1, x2, wt, bt, w1, b1, g1, be1, m1, v1, w2, b2, g2, be2, m2, v2):
    N, C_in, H1, W1 = x1.shape
    _, C_half, H, W = x2.shape
    H_up, W_up = 2 * H1, 2 * W1
    oY, oX = (H - H_up) // 2, (W - W_up) // 2
    f32, bf16 = jnp.float32, jnp.bfloat16

    eps = 1e-5
    s1 = g1 / jnp.sqrt(v1 + eps)
    w1e = w1 * s1[:, None, None, None]
    b1e = (b1 - m1) * s1 + be1
    s2 = g2 / jnp.sqrt(v2 + eps)
    w2e = w2 * s2[:, None, None, None]
    b2e = (b2 - m2) * s2 + be2
    C_mid, C_cat = w1e.shape[0], w1e.shape[1]
    C_out = w2e.shape[0]

    body, Wx, band, SxL, offs = _make_fused_kernel(
        H1, W1, C_in, H, W, H_up, W_up, oY, oX, C_half, C_mid, C_out)

    w1r = jnp.transpose(w1e, (2, 3, 0, 1)).reshape(9 * C_mid, C_cat)
    w2r = jnp.transpose(w2e, (2, 3, 0, 1)).reshape(9 * C_out, C_mid)
    w2r = jnp.pad(w2r, ((0, 0), (0, C_cat - C_mid)))
    wun = jnp.transpose(wt, (0, 2, 3, 1)).reshape(C_in, 2, 2 * C_half)
    wun = jnp.transpose(wun, (1, 0, 2)).reshape(2 * C_in, 2 * C_half)
    bt4 = jnp.tile(bt, 4).reshape(2, 2 * C_half)
    wa = jnp.concatenate([w1r, w2r, wun, bt4], axis=0).astype(bf16)
    pad_rows = (-wa.shape[0]) % 8
    wa = jnp.pad(wa, ((0, pad_rows), (0, 0)))

    fb = jnp.concatenate([b1e, b2e], axis=0).reshape(C_mid + C_out, 1)
    fb = jnp.tile(fb, (1, 128)).astype(f32)

    x1r = x1.reshape(N, C_in, H1 * W1)
    x2r = x2.reshape(N, C_half, H * W)

    smp = lambda n: (n, 0, 0)
    cst2 = lambda n: (0, 0)
    out_flat = pl.pallas_call(
        body,
        out_shape=jax.ShapeDtypeStruct((N, C_out, H * W), f32),
        grid=(N,),
        in_specs=[
            pl.BlockSpec((1, C_in, H1 * W1), smp),
            pl.BlockSpec((1, C_half, H * W), smp),
            pl.BlockSpec(wa.shape, cst2),
            pl.BlockSpec(fb.shape, cst2),
        ],
        out_specs=pl.BlockSpec((1, C_out, H * W), smp),
        scratch_shapes=[
            pltpu.VMEM((2 * C_half, SxL), bf16),
            pltpu.VMEM((C_mid, SxL), bf16),
            pltpu.VMEM((H1 * W1, 4 * C_half), f32),
            pltpu.VMEM((SxL, C_half), bf16),
        ],
        compiler_params=pltpu.CompilerParams(
            dimension_semantics=("parallel",),
            vmem_limit_bytes=64 * 1024 * 1024),
    )(x1r, x2r, wa, fb)

    return out_flat.reshape(N, C_out, H, W)
